# baseline (speedup 1.0000x reference)
import functools

import jax
import jax.numpy as jnp
from jax import lax
from jax.experimental import pallas as pl
from jax.experimental.pallas import tpu as pltpu
from jax.experimental.pallas import tpu_sc as plsc

_SMOOTH_W = 0.0008
_SPARSE_W = 0.0008

_L = 16
_NC = 2
_NS = 16
_NW = _NC * _NS
_B = 128
_N = 32768
_RPW = _B // _NW
_UNROLL = 8

_BAG_SCALE = -1.0 / 3.0

_TC_ROWS = 8


def _log1p_newton(z):
    w = 1.0 + z
    t = z * (1.0 - z * (0.5 - z * (1.0 / 3.0)))
    for _ in range(3):
        t = t - 1.0 + w * jnp.exp(-t)
    return t


def _sc_body(logits_hbm, label_hbm, out_hbm, buf0, buf1, label_v, out_v,
             sem0, sem1, lsem):
    cid = lax.axis_index("c")
    sid = lax.axis_index("s")
    wid = sid * _NC + cid
    lanes = lax.iota(jnp.int32, _L)

    bufs = [buf0, buf1]
    sems = [sem0, sem1]
    row0 = wid * _RPW
    copies = [pltpu.async_copy(logits_hbm.at[row0], bufs[0], sems[0])]
    label_copy = pltpu.async_copy(label_hbm, label_v, lsem)

    pos_inf = jnp.full((_L,), jnp.inf, dtype=jnp.float32)
    pos_inf16 = jnp.full((2 * _L,), jnp.inf, dtype=jnp.bfloat16)
    zeros = jnp.zeros((_L,), jnp.float32)
    bag_acc = zeros

    for j in range(_RPW):
        if j + 1 < _RPW:
            copies.append(pltpu.async_copy(
                logits_hbm.at[row0 + j + 1], bufs[(j + 1) % 2],
                sems[(j + 1) % 2]))
        copies[j].wait()
        rbuf = bufs[j % 2]

        @plsc.parallel_loop(
            0, _N, step=2 * _L, unroll=_UNROLL,
            carry=(pos_inf16, pos_inf16, pos_inf16))
        def _row_loop(i, carry, rbuf=rbuf):
            t1, t2, t3 = carry
            a = rbuf[pl.ds(i, _L)]
            b = rbuf[pl.ds(i + _L, _L)]
            z = -plsc.pack(a, b, format=plsc.PackFormat.INTERLEAVED)
            m1 = jnp.maximum(t1, z)
            t1 = jnp.minimum(t1, z)
            m2 = jnp.maximum(t2, m1)
            t2 = jnp.minimum(t2, m1)
            t3 = jnp.minimum(t3, m2)
            return (t1, t2, t3)

        t1b, t2b, t3b = _row_loop

        a1, b1 = plsc.unpack(t1b, format=plsc.PackFormat.INTERLEAVED)
        a2, b2 = plsc.unpack(t2b, format=plsc.PackFormat.INTERLEAVED)
        a3, b3 = plsc.unpack(t3b, format=plsc.PackFormat.INTERLEAVED)
        x1 = jnp.maximum(a1, b1)
        t1 = jnp.minimum(a1, b1)
        y2 = jnp.minimum(a2, b2)
        z2 = jnp.maximum(a2, b2)
        t2 = jnp.minimum(x1, y2)
        t3 = jnp.minimum(jnp.minimum(z2, jnp.maximum(x1, y2)),
                         jnp.minimum(a3, b3))

        gsum = jnp.float32(0.0)
        for _ in range(3):
            g = jnp.min(t1)
            gsum = gsum + g
            gv = jnp.full((_L,), g)
            hit = lanes == plsc.all_reduce_ffs(t1 == gv)
            t1 = jnp.where(hit, t2, t1)
            t2 = jnp.where(hit, t3, t2)
            t3 = jnp.where(hit, pos_inf, t3)
        bag = gsum * _BAG_SCALE
        bag_acc = jnp.where(lanes == j, bag, bag_acc)

    label_copy.wait()
    y = plsc.load_gather(label_v, [row0 + jnp.minimum(lanes, _RPW - 1)])
    b = bag_acc
    ce_vec = jnp.maximum(b, 0.0) - b * y + _log1p_newton(jnp.exp(-jnp.abs(b)))
    ce_vec = jnp.where(lanes < _RPW, ce_vec, zeros)
    ce_p = jnp.sum(ce_vec) * (1.0 / _B)

    out_v[...] = jnp.where(lanes == 0, ce_p, zeros)
    pltpu.sync_copy(out_v, out_hbm.at[wid])


def _sc_topk_ce(logits, label):
    return pl.kernel(
        _sc_body,
        out_type=jax.ShapeDtypeStruct((_NW, _L), jnp.float32),
        mesh=plsc.VectorSubcoreMesh(core_axis_name="c", subcore_axis_name="s"),
        compiler_params=pltpu.CompilerParams(needs_layout_passes=False),
        scratch_types=[
            pltpu.VMEM((_N,), jnp.float32),
            pltpu.VMEM((_N,), jnp.float32),
            pltpu.VMEM((_B,), jnp.float32),
            pltpu.VMEM((_L,), jnp.float32),
            pltpu.SemaphoreType.DMA,
            pltpu.SemaphoreType.DMA,
            pltpu.SemaphoreType.DMA,
        ],
    )(logits, label)


def _tc_body(x_ref, o_ref, sm_acc, sp_acc):
    i = pl.program_id(0)
    x = x_ref[...]
    s = 1.0 / (1.0 + jnp.exp(-x))
    s_next = pltpu.roll(s, _N - 1, 1)
    col = jax.lax.broadcasted_iota(jnp.int32, (_TC_ROWS, _N), 1)
    d = jnp.where(col < _N - 1, s_next - s, 0.0)
    sm_part = (d * d).reshape(_TC_ROWS, _N // 128, 128).sum(axis=1)
    sp_part = s.reshape(_TC_ROWS, _N // 128, 128).sum(axis=1)

    @pl.when(i == 0)
    def _():
        sm_acc[...] = jnp.zeros_like(sm_acc)
        sp_acc[...] = jnp.zeros_like(sp_acc)

    sm_acc[...] = sm_acc[...] + sm_part
    sp_acc[...] = sp_acc[...] + sp_part

    @pl.when(i == _B // _TC_ROWS - 1)
    def _():
        o_ref[0] = jnp.sum(sm_acc[...])
        o_ref[1] = jnp.sum(sp_acc[...])


def _tc_sums(logits):
    return pl.pallas_call(
        _tc_body,
        grid=(_B // _TC_ROWS,),
        in_specs=[pl.BlockSpec((_TC_ROWS, _N), lambda i: (i, 0))],
        out_specs=pl.BlockSpec(memory_space=pltpu.SMEM),
        out_shape=jax.ShapeDtypeStruct((2,), jnp.float32),
        scratch_shapes=[
            pltpu.VMEM((_TC_ROWS, 128), jnp.float32),
            pltpu.VMEM((_TC_ROWS, 128), jnp.float32),
        ],
    )(logits)


@jax.jit
def _run(logits, label):
    sc_out = _sc_topk_ce(logits, label.astype(jnp.float32))
    tc_out = _tc_sums(logits)
    ce = jnp.sum(sc_out[:, 0])
    smooth = tc_out[0] * (1.0 / (_B * (_N - 1)))
    sparse = tc_out[1] * (1.0 / (_B * _N))
    total = ce + _SMOOTH_W * smooth + _SPARSE_W * sparse
    return (total, ce, smooth, sparse)


def kernel(logits, label):
    return _run(logits, label)


# speedup vs baseline: 1.0691x; 1.0691x over previous
import functools

import jax
import jax.numpy as jnp
from jax import lax
from jax.experimental import pallas as pl
from jax.experimental.pallas import tpu as pltpu
from jax.experimental.pallas import tpu_sc as plsc

_SMOOTH_W = 0.0008
_SPARSE_W = 0.0008

_L = 16
_NC = 2
_NS = 16
_NW = _NC * _NS
_B = 128
_N = 32768
_RPW = _B // _NW
_UNROLL = 8

_BAG_SCALE = -1.0 / 3.0

_TC_ROWS = 16


def _log1p_newton(z):
    w = 1.0 + z
    t = z * (1.0 - z * (0.5 - z * (1.0 / 3.0)))
    for _ in range(3):
        t = t - 1.0 + w * jnp.exp(-t)
    return t


def _sc_body(logits_hbm, label_hbm, out_hbm, buf0, buf1, label_v, out_v,
             sem0, sem1, lsem):
    cid = lax.axis_index("c")
    sid = lax.axis_index("s")
    wid = sid * _NC + cid
    lanes = lax.iota(jnp.int32, _L)

    bufs = [buf0, buf1]
    sems = [sem0, sem1]
    row0 = wid * _RPW
    copies = [pltpu.async_copy(logits_hbm.at[row0], bufs[0], sems[0])]
    label_copy = pltpu.async_copy(label_hbm, label_v, lsem)

    pos_inf = jnp.full((_L,), jnp.inf, dtype=jnp.float32)
    pos_inf16 = jnp.full((2 * _L,), jnp.inf, dtype=jnp.bfloat16)
    zeros = jnp.zeros((_L,), jnp.float32)
    bag_acc = zeros

    for j in range(_RPW):
        if j + 1 < _RPW:
            copies.append(pltpu.async_copy(
                logits_hbm.at[row0 + j + 1], bufs[(j + 1) % 2],
                sems[(j + 1) % 2]))
        copies[j].wait()
        rbuf = bufs[j % 2]

        @plsc.parallel_loop(
            0, _N, step=2 * _L, unroll=_UNROLL,
            carry=(pos_inf16, pos_inf16, pos_inf16))
        def _row_loop(i, carry, rbuf=rbuf):
            t1, t2, t3 = carry
            a = rbuf[pl.ds(i, _L)]
            b = rbuf[pl.ds(i + _L, _L)]
            z = -plsc.pack(a, b, format=plsc.PackFormat.INTERLEAVED)
            m1 = jnp.maximum(t1, z)
            t1 = jnp.minimum(t1, z)
            m2 = jnp.maximum(t2, m1)
            t2 = jnp.minimum(t2, m1)
            t3 = jnp.minimum(t3, m2)
            return (t1, t2, t3)

        t1b, t2b, t3b = _row_loop

        a1, b1 = plsc.unpack(t1b, format=plsc.PackFormat.INTERLEAVED)
        a2, b2 = plsc.unpack(t2b, format=plsc.PackFormat.INTERLEAVED)
        a3, b3 = plsc.unpack(t3b, format=plsc.PackFormat.INTERLEAVED)
        x1 = jnp.maximum(a1, b1)
        t1 = jnp.minimum(a1, b1)
        y2 = jnp.minimum(a2, b2)
        z2 = jnp.maximum(a2, b2)
        t2 = jnp.minimum(x1, y2)
        t3 = jnp.minimum(jnp.minimum(z2, jnp.maximum(x1, y2)),
                         jnp.minimum(a3, b3))

        gsum = jnp.float32(0.0)
        for _ in range(3):
            g = jnp.min(t1)
            gsum = gsum + g
            gv = jnp.full((_L,), g)
            hit = lanes == plsc.all_reduce_ffs(t1 == gv)
            t1 = jnp.where(hit, t2, t1)
            t2 = jnp.where(hit, t3, t2)
            t3 = jnp.where(hit, pos_inf, t3)
        bag = gsum * _BAG_SCALE
        bag_acc = jnp.where(lanes == j, bag, bag_acc)

    label_copy.wait()
    y = plsc.load_gather(label_v, [row0 + jnp.minimum(lanes, _RPW - 1)])
    b = bag_acc
    ce_vec = jnp.maximum(b, 0.0) - b * y + _log1p_newton(jnp.exp(-jnp.abs(b)))
    ce_vec = jnp.where(lanes < _RPW, ce_vec, zeros)
    ce_p = jnp.sum(ce_vec) * (1.0 / _B)

    out_v[...] = jnp.where(lanes == 0, ce_p, zeros)
    pltpu.sync_copy(out_v, out_hbm.at[wid])


def _sc_topk_ce(logits, label):
    return pl.kernel(
        _sc_body,
        out_type=jax.ShapeDtypeStruct((_NW, _L), jnp.float32),
        mesh=plsc.VectorSubcoreMesh(core_axis_name="c", subcore_axis_name="s"),
        compiler_params=pltpu.CompilerParams(needs_layout_passes=False),
        scratch_types=[
            pltpu.VMEM((_N,), jnp.float32),
            pltpu.VMEM((_N,), jnp.float32),
            pltpu.VMEM((_B,), jnp.float32),
            pltpu.VMEM((_L,), jnp.float32),
            pltpu.SemaphoreType.DMA,
            pltpu.SemaphoreType.DMA,
            pltpu.SemaphoreType.DMA,
        ],
    )(logits, label)


def _tc_body(x_ref, o_ref, sm_acc, sp_acc):
    i = pl.program_id(0)
    x = x_ref[...]
    s = 1.0 / (1.0 + jnp.exp(-x))
    s_next = pltpu.roll(s, _N - 1, 1)
    col = jax.lax.broadcasted_iota(jnp.int32, (_TC_ROWS, _N), 1)
    d = jnp.where(col < _N - 1, s_next - s, 0.0)
    sm_part = (d * d).reshape(_TC_ROWS, _N // 128, 128).sum(axis=1)
    sp_part = s.reshape(_TC_ROWS, _N // 128, 128).sum(axis=1)

    @pl.when(i == 0)
    def _():
        sm_acc[...] = jnp.zeros_like(sm_acc)
        sp_acc[...] = jnp.zeros_like(sp_acc)

    sm_acc[...] = sm_acc[...] + sm_part
    sp_acc[...] = sp_acc[...] + sp_part

    @pl.when(i == _B // _TC_ROWS - 1)
    def _():
        o_ref[0] = jnp.sum(sm_acc[...])
        o_ref[1] = jnp.sum(sp_acc[...])


def _tc_sums(logits):
    return pl.pallas_call(
        _tc_body,
        grid=(_B // _TC_ROWS,),
        in_specs=[pl.BlockSpec((_TC_ROWS, _N), lambda i: (i, 0))],
        out_specs=pl.BlockSpec(memory_space=pltpu.SMEM),
        out_shape=jax.ShapeDtypeStruct((2,), jnp.float32),
        scratch_shapes=[
            pltpu.VMEM((_TC_ROWS, 128), jnp.float32),
            pltpu.VMEM((_TC_ROWS, 128), jnp.float32),
        ],
    )(logits)


@jax.jit
def _run(logits, label):
    sc_out = _sc_topk_ce(logits, label.astype(jnp.float32))
    tc_out = _tc_sums(logits)
    ce = jnp.sum(sc_out[:, 0])
    smooth = tc_out[0] * (1.0 / (_B * (_N - 1)))
    sparse = tc_out[1] * (1.0 / (_B * _N))
    total = ce + _SMOOTH_W * smooth + _SPARSE_W * sparse
    return (total, ce, smooth, sparse)


def kernel(logits, label):
    return _run(logits, label)
